# Initial kernel scaffold; baseline (speedup 1.0000x reference)
import jax
import jax.numpy as jnp
from jax.experimental import pallas as pl


def kernel(queries, keys, values, attn_mask):
    raise NotImplementedError("write your pallas kernel here")


# speedup vs baseline: 1.3974x; 1.3974x over previous
import math

import jax
import jax.numpy as jnp
from jax.experimental import pallas as pl
from jax.experimental.pallas import tpu as pltpu

_U = 40
_TILE = 512

_NEG = -1e30


def _head_kernel(idx_ref, q_ref, k_ref, v_ref,
                 cs_ref, tops_ref, ctx_ref,
                 cnt_ref, m_ref, qtop_ref):
    h = pl.program_id(0)
    L, D = q_ref.shape[2], q_ref.shape[3]
    nt = L // _TILE

    @pl.when(h == 0)
    def _build_cnt():
        kio = jax.lax.broadcasted_iota(jnp.int32, (_TILE, L), 1)

        def cbody(t, carry):
            base = t * _TILE
            idx_t = idx_ref[pl.ds(base, _TILE), :]
            acc = jnp.zeros((_TILE, L), jnp.float32)
            for j in range(_U):
                acc += (kio == idx_t[:, j:j + 1]).astype(jnp.float32)
            cnt_ref[pl.ds(base, _TILE), :] = acc
            return carry

        jax.lax.fori_loop(0, nt, cbody, 0)

    Q = q_ref[0, 0]
    K = k_ref[0, 0]
    V = v_ref[0, 0]

    dn_t = (((1,), (1,)), ((), ()))
    dn_n = (((1,), (0,)), ((), ()))
    dn_c0 = (((0,), (0,)), ((), ()))

    def s1body(t, carry):
        base = t * _TILE
        Qt = q_ref[0, 0, pl.ds(base, _TILE), :]
        cnt_t = cnt_ref[pl.ds(base, _TILE), :]
        S = jax.lax.dot_general(Qt, K, dn_t,
                                preferred_element_type=jnp.float32, precision=jax.lax.Precision.HIGHEST)
        KSt = jax.lax.dot_general(cnt_t, K, dn_n,
                                  preferred_element_type=jnp.float32, precision=jax.lax.Precision.HIGHEST)
        ssum = jnp.sum(Qt * KSt, axis=1, keepdims=True)
        smax = jnp.max(jnp.where(cnt_t > 0.0, S, _NEG),
                       axis=1, keepdims=True)
        m_ref[pl.ds(base, _TILE), :] = smax - ssum * (1.0 / _U)
        return carry

    jax.lax.fori_loop(0, nt, s1body, 0)

    qio = jax.lax.broadcasted_iota(jnp.int32, (L, 1), 0)

    def _topk_body(i, M):
        mx = jnp.max(M)
        sel = jnp.max(jnp.where(M == mx, qio, -1))
        oh = (qio == sel).astype(jnp.float32)
        row = jax.lax.dot_general(oh, Q, dn_c0,
                                  preferred_element_type=jnp.float32, precision=jax.lax.Precision.HIGHEST)
        qtop_ref[pl.ds(i, 1), :] = row
        tops_ref[0, pl.ds(i, 1), :] = jnp.full((1, 1), sel, jnp.int32)
        return jnp.where(qio == sel, _NEG, M)

    jax.lax.fori_loop(0, _U, _topk_body, m_ref[:, :])

    Qtop = qtop_ref[:, :]
    pos_col = tops_ref[0, :, :]

    S2 = jax.lax.dot_general(Qtop, K, dn_t,
                             preferred_element_type=jnp.float32, precision=jax.lax.Precision.HIGHEST)
    S2 = S2 * (1.0 / math.sqrt(D))
    kio2 = jax.lax.broadcasted_iota(jnp.int32, (_U, L), 1)
    S2 = jnp.where(kio2 > pos_col, -jnp.inf, S2)
    S2 = S2 - jnp.max(S2, axis=1, keepdims=True)
    E = jnp.exp(S2)
    A = E / jnp.sum(E, axis=1, keepdims=True)
    ctx_ref[0] = jax.lax.dot_general(A, V, dn_n,
                                     preferred_element_type=jnp.float32, precision=jax.lax.Precision.HIGHEST)

    rio = jax.lax.broadcasted_iota(jnp.int32, (_TILE, _TILE), 0)
    cio = jax.lax.broadcasted_iota(jnp.int32, (_TILE, _TILE), 1)
    tri = (rio >= cio).astype(jnp.float32)

    def csbody(t, carry):
        base = t * _TILE
        Vt = v_ref[0, 0, pl.ds(base, _TILE), :]
        cs = jax.lax.dot_general(tri, Vt, dn_n,
                                 preferred_element_type=jnp.float32, precision=jax.lax.Precision.HIGHEST) + carry
        cs_ref[0, pl.ds(base, _TILE), :] = cs
        return cs[_TILE - 1:_TILE, :]

    jax.lax.fori_loop(0, nt, csbody, jnp.zeros((1, D), jnp.float32))


def _merge_kernel(tops_ref, ctx_ref, cs_ref, out_ref, buf_ref, flag_ref):
    h = pl.program_id(0)
    L = cs_ref.shape[1]
    n = ctx_ref.shape[0]

    @pl.when(h == 0)
    def _scatter():
        flag_ref[:, :] = jnp.zeros((L, 1), jnp.float32)
        for t in range(n):
            p = tops_ref[t, 0]
            buf_ref[pl.ds(p, 1), :] = ctx_ref[t:t + 1, :]
            flag_ref[pl.ds(p, 1), :] = jnp.ones((1, 1), jnp.float32)

    merged = jnp.where(flag_ref[:, :] > 0.0, buf_ref[:, :], cs_ref[0])
    out_ref[0] = merged


@jax.jit
def kernel(queries, keys, values, attn_mask):
    B, H, L, D = queries.shape
    idx = jax.random.randint(jax.random.key(42), (L, _U), 0, L)

    cs, tops, ctx = pl.pallas_call(
        _head_kernel,
        grid=(H,),
        in_specs=[
            pl.BlockSpec((L, _U), lambda h: (0, 0)),
            pl.BlockSpec((1, 1, L, D), lambda h: (0, h, 0, 0)),
            pl.BlockSpec((1, 1, L, D), lambda h: (0, h, 0, 0)),
            pl.BlockSpec((1, 1, L, D), lambda h: (0, h, 0, 0)),
        ],
        out_specs=[
            pl.BlockSpec((1, L, D), lambda h: (h, 0, 0)),
            pl.BlockSpec((1, _U, 1), lambda h: (h, 0, 0)),
            pl.BlockSpec((1, _U, D), lambda h: (h, 0, 0)),
        ],
        out_shape=[
            jax.ShapeDtypeStruct((H, L, D), jnp.float32),
            jax.ShapeDtypeStruct((H, _U, 1), jnp.int32),
            jax.ShapeDtypeStruct((H, _U, D), jnp.float32),
        ],
        scratch_shapes=[
            pltpu.VMEM((L, L), jnp.float32),
            pltpu.VMEM((L, 1), jnp.float32),
            pltpu.VMEM((_U, D), jnp.float32),
        ],
    )(idx, queries, keys, values)

    tops_flat = tops.reshape(H * _U, 1)
    ctx_flat = ctx.reshape(H * _U, D)

    out = pl.pallas_call(
        _merge_kernel,
        grid=(H,),
        in_specs=[
            pl.BlockSpec((H * _U, 1), lambda h: (0, 0)),
            pl.BlockSpec((H * _U, D), lambda h: (0, 0)),
            pl.BlockSpec((1, L, D), lambda h: (h, 0, 0)),
        ],
        out_specs=pl.BlockSpec((1, L, D), lambda h: (h, 0, 0)),
        out_shape=jax.ShapeDtypeStruct((H, L, D), jnp.float32),
        scratch_shapes=[
            pltpu.VMEM((L, D), jnp.float32),
            pltpu.VMEM((L, 1), jnp.float32),
        ],
    )(tops_flat, ctx_flat, cs)
    return jnp.transpose(out, (1, 0, 2))[None]


# speedup vs baseline: 1.8321x; 1.3111x over previous
import math

import jax
import jax.numpy as jnp
from jax.experimental import pallas as pl
from jax.experimental.pallas import tpu as pltpu

_U = 40
_TILE = 512

_NEG = -1e30


def _head_kernel(idx_ref, q_ref, k_ref, v_ref,
                 cs_ref, tops_ref, ctx_ref,
                 cnt_ref, m_ref, qtop_ref):
    h = pl.program_id(0)
    L, D = q_ref.shape[2], q_ref.shape[3]
    nt = L // _TILE

    @pl.when(h == 0)
    def _build_cnt():
        kio = jax.lax.broadcasted_iota(jnp.int32, (_TILE, L), 1)

        def cbody(t, carry):
            base = t * _TILE
            idx_t = idx_ref[pl.ds(base, _TILE), :]
            acc = jnp.zeros((_TILE, L), jnp.float32)
            for j in range(_U):
                acc += (kio == idx_t[:, j:j + 1]).astype(jnp.float32)
            cnt_ref[pl.ds(base, _TILE), :] = acc.astype(jnp.int8)
            return carry

        jax.lax.fori_loop(0, nt, cbody, 0)

    Q = q_ref[0, 0]
    K = k_ref[0, 0]
    V = v_ref[0, 0]

    dn_t = (((1,), (1,)), ((), ()))
    dn_n = (((1,), (0,)), ((), ()))
    dn_c0 = (((0,), (0,)), ((), ()))

    def s1body(t, carry):
        base = t * _TILE
        Qt = q_ref[0, 0, pl.ds(base, _TILE), :]
        cnt_t = cnt_ref[pl.ds(base, _TILE), :].astype(jnp.float32)
        S = jax.lax.dot_general(Qt, K, dn_t,
                                preferred_element_type=jnp.float32)
        ssum = jnp.sum(S * cnt_t, axis=1, keepdims=True)
        smax = jnp.max(jnp.where(cnt_t > 0.0, S, _NEG),
                       axis=1, keepdims=True)
        m_ref[pl.ds(base, _TILE), :] = smax - ssum * (1.0 / _U)
        return carry

    jax.lax.fori_loop(0, nt, s1body, 0)

    qio = jax.lax.broadcasted_iota(jnp.int32, (L, 1), 0)

    def _topk_body(i, M):
        mx = jnp.max(M)
        sel = jnp.max(jnp.where(M == mx, qio, -1))
        oh = (qio == sel).astype(jnp.float32)
        row = jax.lax.dot_general(oh, Q, dn_c0,
                                  preferred_element_type=jnp.float32, precision=jax.lax.Precision.HIGHEST)
        qtop_ref[pl.ds(i, 1), :] = row
        tops_ref[0, pl.ds(i, 1), :] = jnp.full((1, 1), sel, jnp.int32)
        return jnp.where(qio == sel, _NEG, M)

    jax.lax.fori_loop(0, _U, _topk_body, m_ref[:, :])

    Qtop = qtop_ref[:, :]
    pos_col = tops_ref[0, :, :]

    S2 = jax.lax.dot_general(Qtop, K, dn_t,
                             preferred_element_type=jnp.float32)
    S2 = S2 * (1.0 / math.sqrt(D))
    kio2 = jax.lax.broadcasted_iota(jnp.int32, (_U, L), 1)
    S2 = jnp.where(kio2 > pos_col, -jnp.inf, S2)
    S2 = S2 - jnp.max(S2, axis=1, keepdims=True)
    E = jnp.exp(S2)
    A = E / jnp.sum(E, axis=1, keepdims=True)
    ctx_ref[0] = jax.lax.dot_general(A, V, dn_n,
                                     preferred_element_type=jnp.float32)

    rio = jax.lax.broadcasted_iota(jnp.int32, (_TILE, _TILE), 0)
    cio = jax.lax.broadcasted_iota(jnp.int32, (_TILE, _TILE), 1)
    tri = (rio >= cio).astype(jnp.float32)

    def csbody(t, carry):
        base = t * _TILE
        Vt = v_ref[0, 0, pl.ds(base, _TILE), :]
        cs = jax.lax.dot_general(tri, Vt, dn_n,
                                 preferred_element_type=jnp.float32, precision=jax.lax.Precision.HIGHEST) + carry
        cs_ref[0, pl.ds(base, _TILE), :] = cs
        return cs[_TILE - 1:_TILE, :]

    jax.lax.fori_loop(0, nt, csbody, jnp.zeros((1, D), jnp.float32))


def _merge_kernel(tops_ref, ctx_ref, cs_ref, out_ref, buf_ref, flag_ref):
    h = pl.program_id(0)
    L = cs_ref.shape[1]
    n = ctx_ref.shape[0]

    @pl.when(h == 0)
    def _scatter():
        flag_ref[:, :] = jnp.zeros((L, 1), jnp.float32)
        for t in range(n):
            p = tops_ref[t, 0]
            buf_ref[pl.ds(p, 1), :] = ctx_ref[t:t + 1, :]
            flag_ref[pl.ds(p, 1), :] = jnp.ones((1, 1), jnp.float32)

    merged = jnp.where(flag_ref[:, :] > 0.0, buf_ref[:, :], cs_ref[0])
    out_ref[0] = merged


@jax.jit
def kernel(queries, keys, values, attn_mask):
    B, H, L, D = queries.shape
    idx = jax.random.randint(jax.random.key(42), (L, _U), 0, L)

    cs, tops, ctx = pl.pallas_call(
        _head_kernel,
        grid=(H,),
        in_specs=[
            pl.BlockSpec((L, _U), lambda h: (0, 0)),
            pl.BlockSpec((1, 1, L, D), lambda h: (0, h, 0, 0)),
            pl.BlockSpec((1, 1, L, D), lambda h: (0, h, 0, 0)),
            pl.BlockSpec((1, 1, L, D), lambda h: (0, h, 0, 0)),
        ],
        out_specs=[
            pl.BlockSpec((1, L, D), lambda h: (h, 0, 0)),
            pl.BlockSpec((1, _U, 1), lambda h: (h, 0, 0)),
            pl.BlockSpec((1, _U, D), lambda h: (h, 0, 0)),
        ],
        out_shape=[
            jax.ShapeDtypeStruct((H, L, D), jnp.float32),
            jax.ShapeDtypeStruct((H, _U, 1), jnp.int32),
            jax.ShapeDtypeStruct((H, _U, D), jnp.float32),
        ],
        scratch_shapes=[
            pltpu.VMEM((L, L), jnp.int8),
            pltpu.VMEM((L, 1), jnp.float32),
            pltpu.VMEM((_U, D), jnp.float32),
        ],
    )(idx, queries, keys, values)

    tops_flat = tops.reshape(H * _U, 1)
    ctx_flat = ctx.reshape(H * _U, D)

    out = pl.pallas_call(
        _merge_kernel,
        grid=(H,),
        in_specs=[
            pl.BlockSpec((H * _U, 1), lambda h: (0, 0)),
            pl.BlockSpec((H * _U, D), lambda h: (0, 0)),
            pl.BlockSpec((1, L, D), lambda h: (h, 0, 0)),
        ],
        out_specs=pl.BlockSpec((1, L, D), lambda h: (h, 0, 0)),
        out_shape=jax.ShapeDtypeStruct((H, L, D), jnp.float32),
        scratch_shapes=[
            pltpu.VMEM((L, D), jnp.float32),
            pltpu.VMEM((L, 1), jnp.float32),
        ],
    )(tops_flat, ctx_flat, cs)
    return jnp.transpose(out, (1, 0, 2))[None]
